# baseline (speedup 1.0000x reference)
import functools

import jax
import jax.numpy as jnp
from jax import lax
from jax.experimental import pallas as pl
from jax.experimental.pallas import tpu as pltpu
from jax.experimental.pallas import tpu_sc as plsc

_SHAPES = ((64, 64), (32, 32), (16, 16), (8, 8))
_H = 8
_NL = 4
_P = 4
_D = 256
_DH = 32
_B = 2
_L = sum(h * w for h, w in _SHAPES)
_NQ = _L
_NS = _NL * _P
_SLAB = _L * 32
_NQH = _NQ // 2
_QC = 16
_NCHUNK = _NQH // _QC

_LVL_START = []
_s = 0
for _hh, _ww in _SHAPES:
    _LVL_START.append(_s)
    _s += _hh * _ww

_W_PER_S = [float(w) for (h, w) in _SHAPES for _ in range(_P)]
_H_PER_S = [float(h) for (h, w) in _SHAPES for _ in range(_P)]
_WI_PER_S = [w for (h, w) in _SHAPES for _ in range(_P)]
_HI_PER_S = [h for (h, w) in _SHAPES for _ in range(_P)]
_START_PER_S = [_LVL_START[s // _P] for s in range(_NS)]


def _mm_body(x_ref, w_ref, b_ref, o_ref):
    o_ref[...] = (
        jnp.dot(x_ref[...], w_ref[...], preferred_element_type=jnp.float32)
        + b_ref[...]
    )


def _mm(x, w, b, blk=640):
    m, k = x.shape
    n = w.shape[1]
    grid = (m // blk,)
    return pl.pallas_call(
        _mm_body,
        grid=grid,
        in_specs=[
            pl.BlockSpec((blk, k), lambda i: (i, 0)),
            pl.BlockSpec((k, n), lambda i: (0, 0)),
            pl.BlockSpec((1, n), lambda i: (0, 0)),
        ],
        out_specs=pl.BlockSpec((blk, n), lambda i: (i, 0)),
        out_shape=jax.ShapeDtypeStruct((m, n), jnp.float32),
    )(x, w, b.reshape(1, n))


def _val_body(enc_ref, wlo_ref, whi_ref, blo_ref, bhi_ref, o_ref):
    x = enc_ref[0]
    lo = jnp.dot(x, wlo_ref[0], preferred_element_type=jnp.float32) + blo_ref[0]
    hi = jnp.dot(x, whi_ref[0], preferred_element_type=jnp.float32) + bhi_ref[0]
    lo16 = lax.bitcast_convert_type(lo.astype(jnp.bfloat16),
                                    jnp.uint16).astype(jnp.int32)
    hi16 = lax.bitcast_convert_type(hi.astype(jnp.bfloat16),
                                    jnp.uint16).astype(jnp.int32)
    o_ref[0, 0] = jnp.bitwise_or(lax.shift_left(hi16, 16), lo16)


def _val(enc, wlo, whi, blo, bhi, qb=680):
    nqb = _L // qb
    grid = (_B, _H, nqb)
    return pl.pallas_call(
        _val_body,
        grid=grid,
        in_specs=[
            pl.BlockSpec((1, qb, _D), lambda b, h, q: (b, q, 0)),
            pl.BlockSpec((1, _D, 16), lambda b, h, q: (h, 0, 0)),
            pl.BlockSpec((1, _D, 16), lambda b, h, q: (h, 0, 0)),
            pl.BlockSpec((1, 1, 16), lambda b, h, q: (h, 0, 0)),
            pl.BlockSpec((1, 1, 16), lambda b, h, q: (h, 0, 0)),
        ],
        out_specs=pl.BlockSpec((1, 1, qb, 16), lambda b, h, q: (b, h, q, 0)),
        out_shape=jax.ShapeDtypeStruct((_B, _H, _L, 16), jnp.int32),
    )(enc, wlo, whi, blo, bhi)


def _locs_body(hs_ref, rpx_ref, rpy_ref, wx_ref, wy_ref, wa_ref,
               bx_ref, by_ref, ba_ref, g_ref, fc_ref, ic_ref,
               i0_ref, i1_ref, i2_ref, i3_ref,
               c0_ref, c1_ref, c2_ref, c3_ref, aw_ref):
    hs = hs_ref[0]
    offx = jnp.dot(hs, wx_ref[...], preferred_element_type=jnp.float32) + bx_ref[...]
    offy = jnp.dot(hs, wy_ref[...], preferred_element_type=jnp.float32) + by_ref[...]
    lg = jnp.dot(hs, wa_ref[...], preferred_element_type=jnp.float32) + ba_ref[...]
    e = jnp.exp(lg)
    aw = e / jnp.dot(e, g_ref[...], preferred_element_type=jnp.float32)
    aw_ref[0] = aw

    wv = fc_ref[0]
    hv = fc_ref[1]
    wvi = ic_ref[0]
    hvi = ic_ref[1]
    start = ic_ref[2]

    rpx = jnp.concatenate([rpx_ref[0]] * _H, axis=-1)
    rpy = jnp.concatenate([rpy_ref[0]] * _H, axis=-1)
    x = rpx * wv + offx - 0.5
    y = rpy * hv + offy - 0.5
    x0f = jnp.floor(x)
    y0f = jnp.floor(y)
    wx1 = x - x0f
    wx0 = 1.0 - wx1
    wy1 = y - y0f
    wy0 = 1.0 - wy1
    x0 = x0f.astype(jnp.int32)
    y0 = y0f.astype(jnp.int32)
    x1 = x0 + 1
    y1 = y0 + 1

    irefs = (i0_ref, i1_ref, i2_ref, i3_ref)
    crefs = (c0_ref, c1_ref, c2_ref, c3_ref)
    corners = ((x0, y0, wx0, wy0), (x1, y0, wx1, wy0),
               (x0, y1, wx0, wy1), (x1, y1, wx1, wy1))
    for k, (cx, cy, wx, wy) in enumerate(corners):
        valid = ((cx >= 0) & (cx < wvi) & (cy >= 0) & (cy < hvi))
        cxc = jnp.clip(cx, 0, wvi - 1)
        cyc = jnp.clip(cy, 0, hvi - 1)
        irefs[k][...] = start + cyc * wvi + cxc
        crefs[k][...] = wx * wy * aw * valid.astype(jnp.float32)


def _locs(hs, rpx, rpy, wx, wy, wa, bx, by, ba, g, qb=680):
    nqb = _NQ // qb
    grid = (_B, nqb)
    qspec = pl.BlockSpec((qb, 128), lambda b, q: (b * nqb + q, 0))
    full = lambda shape: pl.BlockSpec(shape, lambda b, q: tuple(0 for _ in shape))
    pc = pl.pallas_call(
        _locs_body,
        grid=grid,
        in_specs=[
            pl.BlockSpec((1, qb, _D), lambda b, q: (b, q, 0)),
            pl.BlockSpec((1, qb, _NS), lambda b, q: (b, q, 0)),
            pl.BlockSpec((1, qb, _NS), lambda b, q: (b, q, 0)),
            full((_D, 128)),
            full((_D, 128)),
            full((_D, 128)),
            full((1, 128)),
            full((1, 128)),
            full((1, 128)),
            full((128, 128)),
            full((2, 128)),
            full((3, 128)),
        ],
        out_specs=[qspec] * 8 + [pl.BlockSpec((1, qb, 128), lambda b, q: (b, q, 0))],
        out_shape=[jax.ShapeDtypeStruct((_B * _NQ, 128), jnp.int32)] * 4
        + [jax.ShapeDtypeStruct((_B * _NQ, 128), jnp.float32)] * 4
        + [jax.ShapeDtypeStruct((_B, _NQ, 128), jnp.float32)],
    )
    fc = jnp.asarray([_W_PER_S * _H, _H_PER_S * _H], jnp.float32)
    ic = jnp.asarray([_WI_PER_S * _H, _HI_PER_S * _H, _START_PER_S * _H],
                     jnp.int32)
    return pc(hs, rpx, rpy, wx, wy, wa, bx, by, ba, g, fc, ic)


def _sc_body(value_hbm, i0, i1, i2, i3, c0, c1, c2, c3, out_hbm,
             value_v, idx_v, cw_v, out_v, sem_a, sem_b, osem_a, osem_b):
    c = lax.axis_index("c")
    s = lax.axis_index("s")
    wid = s * 2 + c
    bh = s
    qh = c
    b = bh // _H
    h = bh % _H
    pltpu.sync_copy(value_hbm.at[pl.ds(bh * (_L * 16), _L * 16)], value_v)
    q0 = qh * _NQH
    irefs = (i0, i1, i2, i3)
    crefs = (c0, c1, c2, c3)
    sems = (sem_a, sem_b)
    osems = (osem_a, osem_b)

    def fire(ci, sl):
        rows = b * _NQ + q0 + ci * _QC
        for k in range(4):
            pltpu.async_copy(irefs[k].at[pl.ds(rows, _QC)], idx_v.at[sl, k],
                             sems[sl])
            pltpu.async_copy(crefs[k].at[pl.ds(rows, _QC)], cw_v.at[sl, k],
                             sems[sl])

    def drain(ci, sl):
        rows = b * _NQ + q0 + ci * _QC
        for k in range(4):
            pltpu.make_async_copy(irefs[k].at[pl.ds(rows, _QC)],
                                  idx_v.at[sl, k], sems[sl]).wait()
            pltpu.make_async_copy(crefs[k].at[pl.ds(rows, _QC)],
                                  cw_v.at[sl, k], sems[sl]).wait()

    def out_slice(ci):
        return out_hbm.at[pl.ds(wid * (_NQH * 32) + ci * (_QC * 32), _QC * 32)]

    def compute(ci, sl):
        @pl.when(ci >= 2)
        def _():
            pltpu.make_async_copy(out_v.at[sl], out_slice(ci - 2),
                                  osems[sl]).wait()

        @plsc.parallel_loop(0, _QC, unroll=4)
        def q_body(q):
            p0 = [jnp.zeros((16,), jnp.float32) for _ in range(8)]
            p1 = [jnp.zeros((16,), jnp.float32) for _ in range(8)]
            for blk in range(4):
                iv = idx_v[sl, blk, q, pl.ds(h * 16, 16)]
                wv = cw_v[sl, blk, q, pl.ds(h * 16, 16)]
                for j in range(16):
                    v = value_v[pl.ds(iv[j] * 16, 16)]
                    a = lax.bitcast_convert_type(lax.shift_left(v, 16),
                                                 jnp.float32)
                    bb = lax.bitcast_convert_type(
                        jnp.bitwise_and(v, jnp.int32(-65536)), jnp.float32)
                    w = wv[j]
                    lane = blk * 2 + (j & 1)
                    p0[lane] = p0[lane] + a * w
                    p1[lane] = p1[lane] + bb * w
            acc0 = ((p0[0] + p0[1]) + (p0[2] + p0[3])) + \
                   ((p0[4] + p0[5]) + (p0[6] + p0[7]))
            acc1 = ((p1[0] + p1[1]) + (p1[2] + p1[3])) + \
                   ((p1[4] + p1[5]) + (p1[6] + p1[7]))
            out_v[sl, pl.ds(q * 32, 16)] = acc0
            out_v[sl, pl.ds(q * 32 + 16, 16)] = acc1
        pltpu.async_copy(out_v.at[sl], out_slice(ci), osems[sl])

    fire(0, 0)

    def pair_body(p, _):
        ci = p * 2
        fire(ci + 1, 1)
        drain(ci, 0)
        compute(ci, 0)

        @pl.when(p < _NCHUNK // 2 - 1)
        def _():
            fire(ci + 2, 0)

        drain(ci + 1, 1)
        compute(ci + 1, 1)
        return 0

    lax.fori_loop(0, _NCHUNK // 2, pair_body, 0)
    pltpu.make_async_copy(out_v.at[0], out_slice(_NCHUNK - 2), osems[0]).wait()
    pltpu.make_async_copy(out_v.at[1], out_slice(_NCHUNK - 1), osems[1]).wait()


@functools.cache
def _get_sc_gather():
    return pl.kernel(
        _sc_body,
        out_type=jax.ShapeDtypeStruct((32 * _NQH * 32,), jnp.float32),
        mesh=plsc.VectorSubcoreMesh(core_axis_name="c", subcore_axis_name="s"),
        scratch_types=[
            pltpu.VMEM((_L * 16,), jnp.int32),
            pltpu.VMEM((2, 4, _QC, 128), jnp.int32),
            pltpu.VMEM((2, 4, _QC, 128), jnp.float32),
            pltpu.VMEM((2, _QC * 32), jnp.float32),
            pltpu.SemaphoreType.DMA,
            pltpu.SemaphoreType.DMA,
            pltpu.SemaphoreType.DMA,
            pltpu.SemaphoreType.DMA,
        ],
    )


def _out_body(*refs):
    xs = refs[:8]
    w_ref, b_ref, o_ref = refs[8], refs[9], refs[10]
    acc = jnp.broadcast_to(b_ref[...], o_ref.shape[1:])
    for h in range(_H):
        acc = acc + jnp.dot(xs[h][0], w_ref[h],
                            preferred_element_type=jnp.float32)
    o_ref[0] = acc


def _out(out_sc, w8, b_out, qb=680):
    nqb = _NQH // qb
    grid = (_B, 2, nqb)

    def mk(h):
        return pl.BlockSpec((1, qb, 32),
                            lambda b, qh, q, h=h: (b * 16 + h * 2 + qh, q, 0))

    pc = pl.pallas_call(
        _out_body,
        grid=grid,
        in_specs=[mk(h) for h in range(_H)] + [
            pl.BlockSpec((_H, 32, _D), lambda b, qh, q: (0, 0, 0)),
            pl.BlockSpec((1, _D), lambda b, qh, q: (0, 0)),
        ],
        out_specs=pl.BlockSpec((1, qb, _D),
                               lambda b, qh, q: (b, qh * nqb + q, 0)),
        out_shape=jax.ShapeDtypeStruct((_B, _NQ, _D), jnp.float32),
    )
    return pc(*([out_sc] * 8 + [w8, b_out.reshape(1, _D)]))


def kernel(hidden_states, encoder_hidden_states, reference_points,
           spatial_shapes, level_start_index, W_value, b_value, W_off, b_off,
           W_attn, b_attn, W_out, b_out):
    B, Nq, d = hidden_states.shape
    L = encoder_hidden_states.shape[1]

    wv4 = W_value.reshape(d, _H, 2, 16)
    wlo = wv4[:, :, 0, :].transpose(1, 0, 2)
    whi = wv4[:, :, 1, :].transpose(1, 0, 2)
    bv4 = b_value.reshape(_H, 2, 16)
    blo = bv4[:, 0].reshape(_H, 1, 16)
    bhi = bv4[:, 1].reshape(_H, 1, 16)
    vt = _val(encoder_hidden_states, wlo, whi, blo, bhi)

    woff = W_off.reshape(d, _H, _NS, 2)
    wx = woff[..., 0].reshape(d, 128)
    wy = woff[..., 1].reshape(d, 128)
    boff = b_off.reshape(_H, _NS, 2)
    bx = boff[..., 0].reshape(1, 128)
    by = boff[..., 1].reshape(1, 128)
    ba = b_attn.reshape(1, 128)
    g = jnp.kron(jnp.eye(_H, dtype=jnp.float32),
                 jnp.ones((_NS, _NS), jnp.float32))

    rp16 = jnp.broadcast_to(reference_points[:, :, :, None, :],
                            (B, Nq, _NL, _P, 2)).reshape(B, Nq, _NS, 2)
    rpx = rp16[..., 0]
    rpy = rp16[..., 1]

    i0, i1, i2, i3, c0, c1, c2, c3, aw128 = _locs(
        hidden_states, rpx, rpy, wx, wy, W_attn, bx, by, ba, g)

    out_sc = _get_sc_gather()(vt.reshape(-1), i0, i1, i2, i3, c0, c1, c2, c3)

    out = _out(out_sc.reshape(32, _NQH, 32), W_out.reshape(_H, 32, d), b_out)
    aw = aw128.reshape(B, Nq, _H, _NL, _P)
    return (out, aw)


# speedup vs baseline: 1.1331x; 1.1331x over previous
import functools

import jax
import jax.numpy as jnp
from jax import lax
from jax.experimental import pallas as pl
from jax.experimental.pallas import tpu as pltpu
from jax.experimental.pallas import tpu_sc as plsc

_SHAPES = ((64, 64), (32, 32), (16, 16), (8, 8))
_H = 8
_NL = 4
_P = 4
_D = 256
_DH = 32
_B = 2
_L = sum(h * w for h, w in _SHAPES)
_NQ = _L
_NS = _NL * _P
_SLAB = _L * 32
_NQH = _NQ // 2
_QC = 16
_NCHUNK = _NQH // _QC

_LVL_START = []
_s = 0
for _hh, _ww in _SHAPES:
    _LVL_START.append(_s)
    _s += _hh * _ww

_W_PER_S = [float(w) for (h, w) in _SHAPES for _ in range(_P)]
_H_PER_S = [float(h) for (h, w) in _SHAPES for _ in range(_P)]
_WI_PER_S = [w for (h, w) in _SHAPES for _ in range(_P)]
_HI_PER_S = [h for (h, w) in _SHAPES for _ in range(_P)]
_START_PER_S = [_LVL_START[s // _P] for s in range(_NS)]


def _mm_body(x_ref, w_ref, b_ref, o_ref):
    o_ref[...] = (
        jnp.dot(x_ref[...], w_ref[...], preferred_element_type=jnp.float32)
        + b_ref[...]
    )


def _mm(x, w, b, blk=640):
    m, k = x.shape
    n = w.shape[1]
    grid = (m // blk,)
    return pl.pallas_call(
        _mm_body,
        grid=grid,
        in_specs=[
            pl.BlockSpec((blk, k), lambda i: (i, 0)),
            pl.BlockSpec((k, n), lambda i: (0, 0)),
            pl.BlockSpec((1, n), lambda i: (0, 0)),
        ],
        out_specs=pl.BlockSpec((blk, n), lambda i: (i, 0)),
        out_shape=jax.ShapeDtypeStruct((m, n), jnp.float32),
    )(x, w, b.reshape(1, n))


def _val_body(enc_ref, wlo_ref, whi_ref, blo_ref, bhi_ref, o_ref):
    x = enc_ref[0]
    lo = jnp.dot(x, wlo_ref[0], preferred_element_type=jnp.float32) + blo_ref[0]
    hi = jnp.dot(x, whi_ref[0], preferred_element_type=jnp.float32) + bhi_ref[0]
    lo16 = lax.bitcast_convert_type(lo.astype(jnp.bfloat16),
                                    jnp.uint16).astype(jnp.int32)
    hi16 = lax.bitcast_convert_type(hi.astype(jnp.bfloat16),
                                    jnp.uint16).astype(jnp.int32)
    o_ref[0, 0] = jnp.bitwise_or(lax.shift_left(hi16, 16), lo16)


def _val(enc, wlo, whi, blo, bhi, qb=680):
    nqb = _L // qb
    grid = (_B, _H, nqb)
    return pl.pallas_call(
        _val_body,
        grid=grid,
        in_specs=[
            pl.BlockSpec((1, qb, _D), lambda b, h, q: (b, q, 0)),
            pl.BlockSpec((1, _D, 16), lambda b, h, q: (h, 0, 0)),
            pl.BlockSpec((1, _D, 16), lambda b, h, q: (h, 0, 0)),
            pl.BlockSpec((1, 1, 16), lambda b, h, q: (h, 0, 0)),
            pl.BlockSpec((1, 1, 16), lambda b, h, q: (h, 0, 0)),
        ],
        out_specs=pl.BlockSpec((1, 1, qb, 16), lambda b, h, q: (b, h, q, 0)),
        out_shape=jax.ShapeDtypeStruct((_B, _H, _L, 16), jnp.int32),
    )(enc, wlo, whi, blo, bhi)


def _locs_body(hs_ref, rpx_ref, rpy_ref, wx_ref, wy_ref, wa_ref,
               bx_ref, by_ref, ba_ref, g_ref, fc_ref, ic_ref,
               i0_ref, i1_ref, i2_ref, i3_ref,
               c0_ref, c1_ref, c2_ref, c3_ref, aw_ref):
    hs = hs_ref[0]
    offx = jnp.dot(hs, wx_ref[...], preferred_element_type=jnp.float32) + bx_ref[...]
    offy = jnp.dot(hs, wy_ref[...], preferred_element_type=jnp.float32) + by_ref[...]
    lg = jnp.dot(hs, wa_ref[...], preferred_element_type=jnp.float32) + ba_ref[...]
    e = jnp.exp(lg)
    aw = e / jnp.dot(e, g_ref[...], preferred_element_type=jnp.float32)
    aw_ref[0] = aw

    wv = fc_ref[0]
    hv = fc_ref[1]
    wvi = ic_ref[0]
    hvi = ic_ref[1]
    start = ic_ref[2]

    rpx = jnp.concatenate([rpx_ref[0]] * _H, axis=-1)
    rpy = jnp.concatenate([rpy_ref[0]] * _H, axis=-1)
    x = rpx * wv + offx - 0.5
    y = rpy * hv + offy - 0.5
    x0f = jnp.floor(x)
    y0f = jnp.floor(y)
    wx1 = x - x0f
    wx0 = 1.0 - wx1
    wy1 = y - y0f
    wy0 = 1.0 - wy1
    x0 = x0f.astype(jnp.int32)
    y0 = y0f.astype(jnp.int32)
    x1 = x0 + 1
    y1 = y0 + 1

    irefs = (i0_ref, i1_ref, i2_ref, i3_ref)
    crefs = (c0_ref, c1_ref, c2_ref, c3_ref)
    corners = ((x0, y0, wx0, wy0), (x1, y0, wx1, wy0),
               (x0, y1, wx0, wy1), (x1, y1, wx1, wy1))
    for k, (cx, cy, wx, wy) in enumerate(corners):
        valid = ((cx >= 0) & (cx < wvi) & (cy >= 0) & (cy < hvi))
        cxc = jnp.clip(cx, 0, wvi - 1)
        cyc = jnp.clip(cy, 0, hvi - 1)
        irefs[k][...] = start + cyc * wvi + cxc
        crefs[k][...] = wx * wy * aw * valid.astype(jnp.float32)


def _locs(hs, rpx, rpy, wx, wy, wa, bx, by, ba, g, qb=680):
    nqb = _NQ // qb
    grid = (_B, nqb)
    qspec = pl.BlockSpec((qb, 128), lambda b, q: (b * nqb + q, 0))
    full = lambda shape: pl.BlockSpec(shape, lambda b, q: tuple(0 for _ in shape))
    pc = pl.pallas_call(
        _locs_body,
        grid=grid,
        in_specs=[
            pl.BlockSpec((1, qb, _D), lambda b, q: (b, q, 0)),
            pl.BlockSpec((1, qb, _NS), lambda b, q: (b, q, 0)),
            pl.BlockSpec((1, qb, _NS), lambda b, q: (b, q, 0)),
            full((_D, 128)),
            full((_D, 128)),
            full((_D, 128)),
            full((1, 128)),
            full((1, 128)),
            full((1, 128)),
            full((128, 128)),
            full((2, 128)),
            full((3, 128)),
        ],
        out_specs=[qspec] * 8 + [pl.BlockSpec((1, qb, 128), lambda b, q: (b, q, 0))],
        out_shape=[jax.ShapeDtypeStruct((_B * _NQ, 128), jnp.int32)] * 4
        + [jax.ShapeDtypeStruct((_B * _NQ, 128), jnp.float32)] * 4
        + [jax.ShapeDtypeStruct((_B, _NQ, 128), jnp.float32)],
    )
    fc = jnp.asarray([_W_PER_S * _H, _H_PER_S * _H], jnp.float32)
    ic = jnp.asarray([_WI_PER_S * _H, _HI_PER_S * _H, _START_PER_S * _H],
                     jnp.int32)
    return pc(hs, rpx, rpy, wx, wy, wa, bx, by, ba, g, fc, ic)


def _sc_body(value_hbm, i0, i1, i2, i3, c0, c1, c2, c3, out_hbm,
             value_v, idx_v, cw_v, out_v, sem_a, sem_b, osem_a, osem_b):
    c = lax.axis_index("c")
    s = lax.axis_index("s")
    wid = s * 2 + c
    bh = s
    qh = c
    b = bh // _H
    h = bh % _H
    pltpu.sync_copy(value_hbm.at[pl.ds(bh * (_L * 16), _L * 16)], value_v)
    q0 = qh * _NQH
    irefs = (i0, i1, i2, i3)
    crefs = (c0, c1, c2, c3)
    sems = (sem_a, sem_b)
    osems = (osem_a, osem_b)

    def fire(ci, sl):
        rows = b * _NQ + q0 + ci * _QC
        for k in range(4):
            pltpu.async_copy(irefs[k].at[pl.ds(rows, _QC)], idx_v.at[sl, k],
                             sems[sl])
            pltpu.async_copy(crefs[k].at[pl.ds(rows, _QC)], cw_v.at[sl, k],
                             sems[sl])

    def drain(ci, sl):
        rows = b * _NQ + q0 + ci * _QC
        for k in range(4):
            pltpu.make_async_copy(irefs[k].at[pl.ds(rows, _QC)],
                                  idx_v.at[sl, k], sems[sl]).wait()
            pltpu.make_async_copy(crefs[k].at[pl.ds(rows, _QC)],
                                  cw_v.at[sl, k], sems[sl]).wait()

    def out_slice(ci):
        return out_hbm.at[pl.ds(wid * (_NQH * 32) + ci * (_QC * 32), _QC * 32)]

    def compute(ci, sl):
        @pl.when(ci >= 2)
        def _():
            pltpu.make_async_copy(out_v.at[sl], out_slice(ci - 2),
                                  osems[sl]).wait()

        @plsc.parallel_loop(0, _QC, unroll=2)
        def q_body(q):
            p0 = [jnp.zeros((16,), jnp.float32) for _ in range(8)]
            p1 = [jnp.zeros((16,), jnp.float32) for _ in range(8)]
            for blk in range(4):
                iv = idx_v[sl, blk, q, pl.ds(h * 16, 16)]
                wv = cw_v[sl, blk, q, pl.ds(h * 16, 16)]
                for j in range(16):
                    v = value_v[pl.ds(iv[j] * 16, 16)]
                    a = lax.bitcast_convert_type(lax.shift_left(v, 16),
                                                 jnp.float32)
                    bb = lax.bitcast_convert_type(
                        jnp.bitwise_and(v, jnp.int32(-65536)), jnp.float32)
                    w = wv[j]
                    lane = blk * 2 + (j & 1)
                    p0[lane] = p0[lane] + a * w
                    p1[lane] = p1[lane] + bb * w
            acc0 = ((p0[0] + p0[1]) + (p0[2] + p0[3])) + \
                   ((p0[4] + p0[5]) + (p0[6] + p0[7]))
            acc1 = ((p1[0] + p1[1]) + (p1[2] + p1[3])) + \
                   ((p1[4] + p1[5]) + (p1[6] + p1[7]))
            out_v[sl, pl.ds(q * 32, 16)] = acc0
            out_v[sl, pl.ds(q * 32 + 16, 16)] = acc1
        pltpu.async_copy(out_v.at[sl], out_slice(ci), osems[sl])

    fire(0, 0)

    def pair_body(p, _):
        ci = p * 2
        fire(ci + 1, 1)
        drain(ci, 0)
        compute(ci, 0)

        @pl.when(p < _NCHUNK // 2 - 1)
        def _():
            fire(ci + 2, 0)

        drain(ci + 1, 1)
        compute(ci + 1, 1)
        return 0

    lax.fori_loop(0, _NCHUNK // 2, pair_body, 0)
    pltpu.make_async_copy(out_v.at[0], out_slice(_NCHUNK - 2), osems[0]).wait()
    pltpu.make_async_copy(out_v.at[1], out_slice(_NCHUNK - 1), osems[1]).wait()


@functools.cache
def _get_sc_gather():
    return pl.kernel(
        _sc_body,
        out_type=jax.ShapeDtypeStruct((32 * _NQH * 32,), jnp.float32),
        mesh=plsc.VectorSubcoreMesh(core_axis_name="c", subcore_axis_name="s"),
        scratch_types=[
            pltpu.VMEM((_L * 16,), jnp.int32),
            pltpu.VMEM((2, 4, _QC, 128), jnp.int32),
            pltpu.VMEM((2, 4, _QC, 128), jnp.float32),
            pltpu.VMEM((2, _QC * 32), jnp.float32),
            pltpu.SemaphoreType.DMA,
            pltpu.SemaphoreType.DMA,
            pltpu.SemaphoreType.DMA,
            pltpu.SemaphoreType.DMA,
        ],
    )


def _out_body(*refs):
    xs = refs[:8]
    w_ref, b_ref, o_ref = refs[8], refs[9], refs[10]
    acc = jnp.broadcast_to(b_ref[...], o_ref.shape[1:])
    for h in range(_H):
        acc = acc + jnp.dot(xs[h][0], w_ref[h],
                            preferred_element_type=jnp.float32)
    o_ref[0] = acc


def _out(out_sc, w8, b_out, qb=680):
    nqb = _NQH // qb
    grid = (_B, 2, nqb)

    def mk(h):
        return pl.BlockSpec((1, qb, 32),
                            lambda b, qh, q, h=h: (b * 16 + h * 2 + qh, q, 0))

    pc = pl.pallas_call(
        _out_body,
        grid=grid,
        in_specs=[mk(h) for h in range(_H)] + [
            pl.BlockSpec((_H, 32, _D), lambda b, qh, q: (0, 0, 0)),
            pl.BlockSpec((1, _D), lambda b, qh, q: (0, 0)),
        ],
        out_specs=pl.BlockSpec((1, qb, _D),
                               lambda b, qh, q: (b, qh * nqb + q, 0)),
        out_shape=jax.ShapeDtypeStruct((_B, _NQ, _D), jnp.float32),
    )
    return pc(*([out_sc] * 8 + [w8, b_out.reshape(1, _D)]))


def kernel(hidden_states, encoder_hidden_states, reference_points,
           spatial_shapes, level_start_index, W_value, b_value, W_off, b_off,
           W_attn, b_attn, W_out, b_out):
    B, Nq, d = hidden_states.shape
    L = encoder_hidden_states.shape[1]

    wv4 = W_value.reshape(d, _H, 2, 16)
    wlo = wv4[:, :, 0, :].transpose(1, 0, 2)
    whi = wv4[:, :, 1, :].transpose(1, 0, 2)
    bv4 = b_value.reshape(_H, 2, 16)
    blo = bv4[:, 0].reshape(_H, 1, 16)
    bhi = bv4[:, 1].reshape(_H, 1, 16)
    vt = _val(encoder_hidden_states, wlo, whi, blo, bhi)

    woff = W_off.reshape(d, _H, _NS, 2)
    wx = woff[..., 0].reshape(d, 128)
    wy = woff[..., 1].reshape(d, 128)
    boff = b_off.reshape(_H, _NS, 2)
    bx = boff[..., 0].reshape(1, 128)
    by = boff[..., 1].reshape(1, 128)
    ba = b_attn.reshape(1, 128)
    g = jnp.kron(jnp.eye(_H, dtype=jnp.float32),
                 jnp.ones((_NS, _NS), jnp.float32))

    rp16 = jnp.broadcast_to(reference_points[:, :, :, None, :],
                            (B, Nq, _NL, _P, 2)).reshape(B, Nq, _NS, 2)
    rpx = rp16[..., 0]
    rpy = rp16[..., 1]

    i0, i1, i2, i3, c0, c1, c2, c3, aw128 = _locs(
        hidden_states, rpx, rpy, wx, wy, W_attn, bx, by, ba, g)

    out_sc = _get_sc_gather()(vt.reshape(-1), i0, i1, i2, i3, c0, c1, c2, c3)

    out = _out(out_sc.reshape(32, _NQH, 32), W_out.reshape(_H, 32, d), b_out)
    aw = aw128.reshape(B, Nq, _H, _NL, _P)
    return (out, aw)


# speedup vs baseline: 1.2648x; 1.1162x over previous
import functools

import jax
import jax.numpy as jnp
from jax import lax
from jax.experimental import pallas as pl
from jax.experimental.pallas import tpu as pltpu
from jax.experimental.pallas import tpu_sc as plsc

_SHAPES = ((64, 64), (32, 32), (16, 16), (8, 8))
_H = 8
_NL = 4
_P = 4
_D = 256
_DH = 32
_B = 2
_L = sum(h * w for h, w in _SHAPES)
_NQ = _L
_NS = _NL * _P
_SLAB = _L * 32
_NQH = _NQ // 2
_QC = 16
_NCHUNK = _NQH // _QC

_LVL_START = []
_s = 0
for _hh, _ww in _SHAPES:
    _LVL_START.append(_s)
    _s += _hh * _ww

_W_PER_S = [float(w) for (h, w) in _SHAPES for _ in range(_P)]
_H_PER_S = [float(h) for (h, w) in _SHAPES for _ in range(_P)]
_WI_PER_S = [w for (h, w) in _SHAPES for _ in range(_P)]
_HI_PER_S = [h for (h, w) in _SHAPES for _ in range(_P)]
_START_PER_S = [_LVL_START[s // _P] for s in range(_NS)]


def _mm_body(x_ref, w_ref, b_ref, o_ref):
    o_ref[...] = (
        jnp.dot(x_ref[...], w_ref[...], preferred_element_type=jnp.float32)
        + b_ref[...]
    )


def _mm(x, w, b, blk=640):
    m, k = x.shape
    n = w.shape[1]
    grid = (m // blk,)
    return pl.pallas_call(
        _mm_body,
        grid=grid,
        in_specs=[
            pl.BlockSpec((blk, k), lambda i: (i, 0)),
            pl.BlockSpec((k, n), lambda i: (0, 0)),
            pl.BlockSpec((1, n), lambda i: (0, 0)),
        ],
        out_specs=pl.BlockSpec((blk, n), lambda i: (i, 0)),
        out_shape=jax.ShapeDtypeStruct((m, n), jnp.float32),
    )(x, w, b.reshape(1, n))


def _val_body(enc_ref, wlo_ref, whi_ref, blo_ref, bhi_ref, o_ref):
    x = enc_ref[0]
    lo = jnp.dot(x, wlo_ref[0], preferred_element_type=jnp.float32) + blo_ref[0]
    hi = jnp.dot(x, whi_ref[0], preferred_element_type=jnp.float32) + bhi_ref[0]
    lo16 = lax.bitcast_convert_type(lo.astype(jnp.bfloat16),
                                    jnp.uint16).astype(jnp.int32)
    hi16 = lax.bitcast_convert_type(hi.astype(jnp.bfloat16),
                                    jnp.uint16).astype(jnp.int32)
    o_ref[0, 0] = jnp.bitwise_or(lax.shift_left(hi16, 16), lo16)


def _val(enc, wlo, whi, blo, bhi, qb=2720):
    nqb = _L // qb
    grid = (_B, _H, nqb)
    return pl.pallas_call(
        _val_body,
        grid=grid,
        in_specs=[
            pl.BlockSpec((1, qb, _D), lambda b, h, q: (b, q, 0)),
            pl.BlockSpec((1, _D, 16), lambda b, h, q: (h, 0, 0)),
            pl.BlockSpec((1, _D, 16), lambda b, h, q: (h, 0, 0)),
            pl.BlockSpec((1, 1, 16), lambda b, h, q: (h, 0, 0)),
            pl.BlockSpec((1, 1, 16), lambda b, h, q: (h, 0, 0)),
        ],
        out_specs=pl.BlockSpec((1, 1, qb, 16), lambda b, h, q: (b, h, q, 0)),
        out_shape=jax.ShapeDtypeStruct((_B, _H, _L, 16), jnp.int32),
    )(enc, wlo, whi, blo, bhi)


def _locs_body(hs_ref, rpx_ref, rpy_ref, wx_ref, wy_ref, wa_ref,
               bx_ref, by_ref, ba_ref, g_ref, fc_ref, ic_ref,
               i0_ref, i1_ref, i2_ref, i3_ref,
               c0_ref, c1_ref, c2_ref, c3_ref, aw_ref):
    hs = hs_ref[0]
    offx = jnp.dot(hs, wx_ref[...], preferred_element_type=jnp.float32) + bx_ref[...]
    offy = jnp.dot(hs, wy_ref[...], preferred_element_type=jnp.float32) + by_ref[...]
    lg = jnp.dot(hs, wa_ref[...], preferred_element_type=jnp.float32) + ba_ref[...]
    e = jnp.exp(lg)
    aw = e / jnp.dot(e, g_ref[...], preferred_element_type=jnp.float32)
    aw_ref[0] = aw

    wv = fc_ref[0]
    hv = fc_ref[1]
    wvi = ic_ref[0]
    hvi = ic_ref[1]
    start = ic_ref[2]

    rpx = jnp.concatenate([rpx_ref[0]] * _H, axis=-1)
    rpy = jnp.concatenate([rpy_ref[0]] * _H, axis=-1)
    x = rpx * wv + offx - 0.5
    y = rpy * hv + offy - 0.5
    x0f = jnp.floor(x)
    y0f = jnp.floor(y)
    wx1 = x - x0f
    wx0 = 1.0 - wx1
    wy1 = y - y0f
    wy0 = 1.0 - wy1
    x0 = x0f.astype(jnp.int32)
    y0 = y0f.astype(jnp.int32)
    x1 = x0 + 1
    y1 = y0 + 1

    irefs = (i0_ref, i1_ref, i2_ref, i3_ref)
    crefs = (c0_ref, c1_ref, c2_ref, c3_ref)
    corners = ((x0, y0, wx0, wy0), (x1, y0, wx1, wy0),
               (x0, y1, wx0, wy1), (x1, y1, wx1, wy1))
    for k, (cx, cy, wx, wy) in enumerate(corners):
        valid = ((cx >= 0) & (cx < wvi) & (cy >= 0) & (cy < hvi))
        cxc = jnp.clip(cx, 0, wvi - 1)
        cyc = jnp.clip(cy, 0, hvi - 1)
        irefs[k][...] = (start + cyc * wvi + cxc) * 16
        crefs[k][...] = wx * wy * aw * valid.astype(jnp.float32)


def _locs(hs, rpx, rpy, wx, wy, wa, bx, by, ba, g, qb=680):
    nqb = _NQ // qb
    grid = (_B, nqb)
    qspec = pl.BlockSpec((qb, 128), lambda b, q: (b * nqb + q, 0))
    full = lambda shape: pl.BlockSpec(shape, lambda b, q: tuple(0 for _ in shape))
    pc = pl.pallas_call(
        _locs_body,
        grid=grid,
        in_specs=[
            pl.BlockSpec((1, qb, _D), lambda b, q: (b, q, 0)),
            pl.BlockSpec((1, qb, _NS), lambda b, q: (b, q, 0)),
            pl.BlockSpec((1, qb, _NS), lambda b, q: (b, q, 0)),
            full((_D, 128)),
            full((_D, 128)),
            full((_D, 128)),
            full((1, 128)),
            full((1, 128)),
            full((1, 128)),
            full((128, 128)),
            full((2, 128)),
            full((3, 128)),
        ],
        out_specs=[qspec] * 8 + [pl.BlockSpec((1, qb, 128), lambda b, q: (b, q, 0))],
        out_shape=[jax.ShapeDtypeStruct((_B * _NQ, 128), jnp.int32)] * 4
        + [jax.ShapeDtypeStruct((_B * _NQ, 128), jnp.float32)] * 4
        + [jax.ShapeDtypeStruct((_B, _NQ, 128), jnp.float32)],
    )
    fc = jnp.asarray([_W_PER_S * _H, _H_PER_S * _H], jnp.float32)
    ic = jnp.asarray([_WI_PER_S * _H, _HI_PER_S * _H, _START_PER_S * _H],
                     jnp.int32)
    return pc(hs, rpx, rpy, wx, wy, wa, bx, by, ba, g, fc, ic)


def _sc_body(value_hbm, i0, i1, i2, i3, c0, c1, c2, c3, out_hbm,
             value_v, idx_v, cw_v, out_v, sem_a, sem_b, osem_a, osem_b):
    c = lax.axis_index("c")
    s = lax.axis_index("s")
    wid = s * 2 + c
    bh = s
    qh = c
    b = bh // _H
    h = bh % _H
    pltpu.sync_copy(value_hbm.at[pl.ds(bh * (_L * 16), _L * 16)], value_v)
    q0 = qh * _NQH
    irefs = (i0, i1, i2, i3)
    crefs = (c0, c1, c2, c3)
    sems = (sem_a, sem_b)
    osems = (osem_a, osem_b)

    def fire(ci, sl):
        rows = b * _NQ + q0 + ci * _QC
        for k in range(4):
            pltpu.async_copy(irefs[k].at[pl.ds(rows, _QC)], idx_v.at[sl, k],
                             sems[sl])
            pltpu.async_copy(crefs[k].at[pl.ds(rows, _QC)], cw_v.at[sl, k],
                             sems[sl])

    def drain(ci, sl):
        rows = b * _NQ + q0 + ci * _QC
        for k in range(4):
            pltpu.make_async_copy(irefs[k].at[pl.ds(rows, _QC)],
                                  idx_v.at[sl, k], sems[sl]).wait()
            pltpu.make_async_copy(crefs[k].at[pl.ds(rows, _QC)],
                                  cw_v.at[sl, k], sems[sl]).wait()

    def out_slice(ci):
        return out_hbm.at[pl.ds(wid * (_NQH * 32) + ci * (_QC * 32), _QC * 32)]

    def compute(ci, sl):
        @pl.when(ci >= 2)
        def _():
            pltpu.make_async_copy(out_v.at[sl], out_slice(ci - 2),
                                  osems[sl]).wait()

        @plsc.parallel_loop(0, _QC, unroll=2)
        def q_body(q):
            p0 = [jnp.zeros((16,), jnp.float32) for _ in range(8)]
            p1 = [jnp.zeros((16,), jnp.float32) for _ in range(8)]
            for blk in range(4):
                iv = idx_v[sl, blk, q, pl.ds(h * 16, 16)]
                wv = cw_v[sl, blk, q, pl.ds(h * 16, 16)]
                for j in range(16):
                    v = value_v[pl.ds(iv[j], 16)]
                    a = lax.bitcast_convert_type(lax.shift_left(v, 16),
                                                 jnp.float32)
                    bb = lax.bitcast_convert_type(
                        jnp.bitwise_and(v, jnp.int32(-65536)), jnp.float32)
                    w = wv[j]
                    lane = blk * 2 + (j & 1)
                    p0[lane] = p0[lane] + a * w
                    p1[lane] = p1[lane] + bb * w
            acc0 = ((p0[0] + p0[1]) + (p0[2] + p0[3])) + \
                   ((p0[4] + p0[5]) + (p0[6] + p0[7]))
            acc1 = ((p1[0] + p1[1]) + (p1[2] + p1[3])) + \
                   ((p1[4] + p1[5]) + (p1[6] + p1[7]))
            out_v[sl, pl.ds(q * 32, 16)] = acc0
            out_v[sl, pl.ds(q * 32 + 16, 16)] = acc1
        pltpu.async_copy(out_v.at[sl], out_slice(ci), osems[sl])

    fire(0, 0)

    def pair_body(p, _):
        ci = p * 2
        fire(ci + 1, 1)
        drain(ci, 0)
        compute(ci, 0)

        @pl.when(p < _NCHUNK // 2 - 1)
        def _():
            fire(ci + 2, 0)

        drain(ci + 1, 1)
        compute(ci + 1, 1)
        return 0

    lax.fori_loop(0, _NCHUNK // 2, pair_body, 0)
    pltpu.make_async_copy(out_v.at[0], out_slice(_NCHUNK - 2), osems[0]).wait()
    pltpu.make_async_copy(out_v.at[1], out_slice(_NCHUNK - 1), osems[1]).wait()


@functools.cache
def _get_sc_gather():
    return pl.kernel(
        _sc_body,
        out_type=jax.ShapeDtypeStruct((32 * _NQH * 32,), jnp.float32),
        mesh=plsc.VectorSubcoreMesh(core_axis_name="c", subcore_axis_name="s"),
        scratch_types=[
            pltpu.VMEM((_L * 16,), jnp.int32),
            pltpu.VMEM((2, 4, _QC, 128), jnp.int32),
            pltpu.VMEM((2, 4, _QC, 128), jnp.float32),
            pltpu.VMEM((2, _QC * 32), jnp.float32),
            pltpu.SemaphoreType.DMA,
            pltpu.SemaphoreType.DMA,
            pltpu.SemaphoreType.DMA,
            pltpu.SemaphoreType.DMA,
        ],
    )


def _out_body(*refs):
    xs = refs[:8]
    w_ref, b_ref, o_ref = refs[8], refs[9], refs[10]
    acc = jnp.broadcast_to(b_ref[...], o_ref.shape[1:])
    for h in range(_H):
        acc = acc + jnp.dot(xs[h][0], w_ref[h],
                            preferred_element_type=jnp.float32)
    o_ref[0] = acc


def _out(out_sc, w8, b_out, qb=680):
    nqb = _NQH // qb
    grid = (_B, 2, nqb)

    def mk(h):
        return pl.BlockSpec((1, qb, 32),
                            lambda b, qh, q, h=h: (b * 16 + h * 2 + qh, q, 0))

    pc = pl.pallas_call(
        _out_body,
        grid=grid,
        in_specs=[mk(h) for h in range(_H)] + [
            pl.BlockSpec((_H, 32, _D), lambda b, qh, q: (0, 0, 0)),
            pl.BlockSpec((1, _D), lambda b, qh, q: (0, 0)),
        ],
        out_specs=pl.BlockSpec((1, qb, _D),
                               lambda b, qh, q: (b, qh * nqb + q, 0)),
        out_shape=jax.ShapeDtypeStruct((_B, _NQ, _D), jnp.float32),
    )
    return pc(*([out_sc] * 8 + [w8, b_out.reshape(1, _D)]))


def kernel(hidden_states, encoder_hidden_states, reference_points,
           spatial_shapes, level_start_index, W_value, b_value, W_off, b_off,
           W_attn, b_attn, W_out, b_out):
    B, Nq, d = hidden_states.shape
    L = encoder_hidden_states.shape[1]

    wv4 = W_value.reshape(d, _H, 2, 16)
    wlo = wv4[:, :, 0, :].transpose(1, 0, 2)
    whi = wv4[:, :, 1, :].transpose(1, 0, 2)
    bv4 = b_value.reshape(_H, 2, 16)
    blo = bv4[:, 0].reshape(_H, 1, 16)
    bhi = bv4[:, 1].reshape(_H, 1, 16)
    vt = _val(encoder_hidden_states, wlo, whi, blo, bhi)

    woff = W_off.reshape(d, _H, _NS, 2)
    wx = woff[..., 0].reshape(d, 128)
    wy = woff[..., 1].reshape(d, 128)
    boff = b_off.reshape(_H, _NS, 2)
    bx = boff[..., 0].reshape(1, 128)
    by = boff[..., 1].reshape(1, 128)
    ba = b_attn.reshape(1, 128)
    g = jnp.kron(jnp.eye(_H, dtype=jnp.float32),
                 jnp.ones((_NS, _NS), jnp.float32))

    rp16 = jnp.broadcast_to(reference_points[:, :, :, None, :],
                            (B, Nq, _NL, _P, 2)).reshape(B, Nq, _NS, 2)
    rpx = rp16[..., 0]
    rpy = rp16[..., 1]

    i0, i1, i2, i3, c0, c1, c2, c3, aw128 = _locs(
        hidden_states, rpx, rpy, wx, wy, W_attn, bx, by, ba, g)

    out_sc = _get_sc_gather()(vt.reshape(-1), i0, i1, i2, i3, c0, c1, c2, c3)

    out = _out(out_sc.reshape(32, _NQH, 32), W_out.reshape(_H, 32, d), b_out)
    aw = aw128.reshape(B, Nq, _H, _NL, _P)
    return (out, aw)
